# baseline (speedup 1.0000x reference)
import functools

import jax
import jax.numpy as jnp
from jax import lax
from jax.experimental import pallas as pl
from jax.experimental.pallas import tpu as pltpu
from jax.experimental.pallas import tpu_sc as plsc

_NC = 2
_NS = 16
_NW = _NC * _NS
_LANES = 16

_CH0 = 104


def _make_pool_kernel(batch, seq, emb, bpw):
    ch1 = seq - _CH0
    egr = emb // _LANES
    inv = 1.0 / float(seq)
    mesh = plsc.VectorSubcoreMesh(
        core_axis_name="c", subcore_axis_name="s",
        num_cores=_NC, num_subcores=_NS,
    )

    def body(xt_hbm, table_hbm, out_hbm, idx_v, rows_v, stage_v, sem0, sem1):
        wid = lax.axis_index("s") * _NC + lax.axis_index("c")
        base = wid * bpw
        pltpu.sync_copy(xt_hbm.at[pl.ds(base, bpw)], idx_v)

        sems = (sem0, sem1)

        def gather(bl, buf):
            pltpu.async_copy(
                table_hbm.at[idx_v.at[bl, pl.ds(0, _CH0)]],
                rows_v.at[buf, pl.ds(0, _CH0)],
                sems[buf],
            )
            pltpu.async_copy(
                table_hbm.at[idx_v.at[bl, pl.ds(_CH0, ch1)]],
                rows_v.at[buf, pl.ds(_CH0, ch1)],
                sems[buf],
            )

        def wait(bl, buf):
            pltpu.make_async_copy(
                table_hbm.at[idx_v.at[bl, pl.ds(0, _CH0)]],
                rows_v.at[buf, pl.ds(0, _CH0)],
                sems[buf],
            ).wait()
            pltpu.make_async_copy(
                table_hbm.at[idx_v.at[bl, pl.ds(_CH0, ch1)]],
                rows_v.at[buf, pl.ds(_CH0, ch1)],
                sems[buf],
            ).wait()

        gather(0, 0)

        def pair_body(i2, _):
            for b in range(2):
                bl = i2 * 2 + b

                @pl.when(bl + 1 < bpw)
                def _():
                    gather(bl + 1, 1 - b)

                wait(bl, b)

                zero = jnp.zeros((_LANES,), jnp.float32)
                def row_body(s, accs):
                    out = []
                    for d in range(egr):
                        a, c = accs[2 * d], accs[2 * d + 1]
                        sl = pl.ds(d * _LANES, _LANES)
                        a = a + rows_v[b, 2 * s, sl]
                        c = c + rows_v[b, 2 * s + 1, sl]
                        out += [a, c]
                    return tuple(out)

                accs = lax.fori_loop(
                    0, seq // 2, row_body, (zero,) * (2 * egr))
                for d in range(egr):
                    stage_v[bl, pl.ds(d * _LANES, _LANES)] = (
                        accs[2 * d] + accs[2 * d + 1]) * inv
            return 0

        lax.fori_loop(0, bpw // 2, pair_body, 0)
        pltpu.sync_copy(stage_v, out_hbm.at[pl.ds(base, bpw)])

    return pl.kernel(
        body,
        out_type=jax.ShapeDtypeStruct((batch, emb), jnp.float32),
        mesh=mesh,
        compiler_params=pltpu.CompilerParams(use_tc_tiling_on_sc=False),
        scratch_types=[
            pltpu.VMEM((bpw, seq), jnp.int32),
            pltpu.VMEM((2, seq, emb), jnp.float32),
            pltpu.VMEM((bpw, emb), jnp.float32),
            pltpu.SemaphoreType.DMA,
            pltpu.SemaphoreType.DMA,
        ],
    )


def _mlp_body(p_ref, w1_ref, b1_ref, w2_ref, b2_ref, o_ref):
    h = jnp.dot(p_ref[...], w1_ref[...], preferred_element_type=jnp.float32)
    h = jnp.maximum(h + b1_ref[...], 0.0)
    o_ref[...] = (
        jnp.dot(h, w2_ref[...], preferred_element_type=jnp.float32)
        + b2_ref[...])


@jax.jit
def kernel(x, table, W1, b1, W2, b2):
    seq, batch = x.shape
    vocab, emb = table.shape
    hid = W1.shape[1]
    cls = W2.shape[1]
    bpw = batch // _NW

    xt = jnp.transpose(x)
    pooled = _make_pool_kernel(batch, seq, emb, bpw)(xt, table)

    cp = 128
    w2p = jnp.pad(W2, ((0, 0), (0, cp - cls)))
    b2p = jnp.pad(b2, (0, cp - cls))

    bm = 512
    out = pl.pallas_call(
        _mlp_body,
        grid=(batch // bm,),
        in_specs=[
            pl.BlockSpec((bm, emb), lambda i: (i, 0)),
            pl.BlockSpec((emb, hid), lambda i: (0, 0)),
            pl.BlockSpec((1, hid), lambda i: (0, 0)),
            pl.BlockSpec((hid, cp), lambda i: (0, 0)),
            pl.BlockSpec((1, cp), lambda i: (0, 0)),
        ],
        out_specs=pl.BlockSpec((bm, cp), lambda i: (i, 0)),
        out_shape=jax.ShapeDtypeStruct((batch, cp), jnp.float32),
    )(pooled, W1, b1[None, :], w2p, b2p[None, :])
    return out[:, :cls]
